# baseline (speedup 1.0000x reference)
import functools

import jax
import jax.numpy as jnp
from jax import lax
from jax.experimental import pallas as pl
from jax.experimental.pallas import tpu as pltpu
from jax.experimental.pallas import tpu_sc as plsc

N = 10000
E = 320000
NPAD = 10240
D_IN = 128

NTILE = 16
NCORE = 2
CHUNK = 128
GRP = 8
NGRP = 20
NCHUNK = GRP * NGRP
EPT = NCHUNK * CHUNK
EPAD = EPT * NTILE
ROWS_PT = NPAD // NTILE


def _make_agg(with_counts):
    W = 64
    mesh = plsc.VectorSubcoreMesh(core_axis_name="c", subcore_axis_name="s")
    nbuf = 4
    npre = 2
    out_type = [jax.ShapeDtypeStruct((2 * NPAD, W), jnp.float32)]
    scratch = [
        pltpu.VMEM_SHARED((NPAD, W), jnp.float32),
        pltpu.VMEM_SHARED((NPAD, W), jnp.float32),
        pltpu.VMEM((2, GRP, CHUNK), jnp.int32),
        pltpu.VMEM((2, GRP, CHUNK), jnp.int32),
        [pltpu.VMEM((CHUNK, W), jnp.float32) for _ in range(nbuf)],
        [pltpu.SemaphoreType.DMA for _ in range(nbuf)],
        [pltpu.SemaphoreType.DMA for _ in range(nbuf)],
        pltpu.SemaphoreType.DMA,
    ]
    if with_counts:
        out_type.append(jax.ShapeDtypeStruct((NPAD, 8), jnp.float32))
        scratch += [
            pltpu.VMEM_SHARED((NPAD, 8), jnp.float32),
            pltpu.VMEM((CHUNK, 8), jnp.float32),
            pltpu.SemaphoreType.DMA,
        ]

    def body_common(table, srct, dstt, zeros_w, out, acc, tbl, sidx, didx,
                    bufs, gsems, ssems, isem, c, t, cnt_stuff):
        r0 = t * ROWS_PT
        pltpu.sync_copy(zeros_w.at[pl.ds(r0, ROWS_PT)],
                        acc.at[pl.ds(r0, ROWS_PT)])
        pltpu.sync_copy(table.at[pl.ds(c * NPAD + r0, ROWS_PT)],
                        tbl.at[pl.ds(r0, ROWS_PT)])
        if cnt_stuff is not None:
            ones8, zeros8, cnt_out, cntacc, onesb, csem = cnt_stuff
            pltpu.sync_copy(zeros8.at[pl.ds(r0, ROWS_PT)],
                            cntacc.at[pl.ds(r0, ROWS_PT)])
            pltpu.sync_copy(ones8, onesb)

        def gather_start(slot, j, b):
            pltpu.async_copy(tbl.at[sidx.at[slot, j]], bufs[b], gsems[b])

        def gather_wait(b):
            pltpu.make_async_copy(tbl.at[sidx.at[0, 0]], bufs[b],
                                  gsems[b]).wait()

        def scatter_start(slot, j, b):
            pltpu.async_copy(bufs[b], acc.at[didx.at[slot, j]], ssems[b],
                             add=True)

        def scatter_wait(b):
            pltpu.make_async_copy(bufs[b], acc.at[didx.at[0, 0]],
                                  ssems[b]).wait()

        def count_start(slot, j):
            pltpu.async_copy(onesb, cntacc.at[didx.at[slot, j]], csem,
                             add=True)

        def count_wait():
            pltpu.make_async_copy(onesb, cntacc.at[didx.at[0, 0]],
                                  csem).wait()

        def idx_start(g):
            off = jnp.minimum(g, NGRP - 1) * GRP
            slot = lax.rem(g, 2)
            pltpu.async_copy(srct.at[t, pl.ds(off, GRP)],
                             sidx.at[slot], isem)
            pltpu.async_copy(dstt.at[t, pl.ds(off, GRP)],
                             didx.at[slot], isem)

        def idx_wait(g):
            slot = lax.rem(g, 2)
            pltpu.make_async_copy(srct.at[t, pl.ds(0, GRP)],
                                  sidx.at[slot], isem).wait()
            pltpu.make_async_copy(dstt.at[t, pl.ds(0, GRP)],
                                  didx.at[slot], isem).wait()

        def run_group(slot, nslot, first):
            for j in range(GRP):
                b = j % nbuf
                gather_wait(b)
                scatter_start(slot, j, b)
                if cnt_stuff is not None:
                    if not (first and j == 0):
                        count_wait()
                    count_start(slot, j)
                nj = j + npre
                if nj < GRP:
                    b2 = nj % nbuf
                    if not (first and nj < nbuf):
                        scatter_wait(b2)
                    gather_start(slot, nj, b2)
            for j in range(npre):
                b2 = j % nbuf
                scatter_wait(b2)
                gather_start(nslot, j, b2)

        idx_start(0)
        idx_wait(0)
        plsc.subcore_barrier()
        for j in range(npre):
            gather_start(0, j, j)

        idx_start(1)
        run_group(0, 1, True)
        idx_wait(1)

        def group(g, carry):
            idx_start(g + 1)
            run_group(lax.rem(g, 2), lax.rem(g + 1, 2), False)
            idx_wait(g + 1)
            return carry

        lax.fori_loop(1, NGRP, group, 0)
        for j in range(npre):
            gather_wait(j % nbuf)
        for k in range(nbuf - npre):
            scatter_wait((GRP - (nbuf - npre) + k) % nbuf)
        if cnt_stuff is not None:
            count_wait()
        plsc.subcore_barrier()
        pltpu.sync_copy(acc.at[pl.ds(r0, ROWS_PT)],
                        out.at[pl.ds(c * NPAD + r0, ROWS_PT)])
        if cnt_stuff is not None:
            @pl.when(c == 0)
            def _():
                pltpu.sync_copy(cntacc.at[pl.ds(r0, ROWS_PT)],
                                cnt_out.at[pl.ds(r0, ROWS_PT)])

    if with_counts:
        def body(table, srct, dstt, zeros_w, ones8, zeros8, out, cnt_out,
                 acc, tbl, sidx, didx, bufs, gsems, ssems, isem,
                 cntacc, onesb, csem):
            c = lax.axis_index("c")
            t = lax.axis_index("s")
            body_common(table, srct, dstt, zeros_w, out, acc, tbl, sidx,
                        didx, bufs, gsems, ssems, isem, c, t,
                        (ones8, zeros8, cnt_out, cntacc, onesb, csem))
    else:
        def body(table, srct, dstt, zeros_w, out,
                 acc, tbl, sidx, didx, bufs, gsems, ssems, isem):
            c = lax.axis_index("c")
            t = lax.axis_index("s")
            body_common(table, srct, dstt, zeros_w, out, acc, tbl, sidx,
                        didx, bufs, gsems, ssems, isem, c, t, None)

    out_type = tuple(out_type) if with_counts else out_type[0]
    return pl.kernel(body, out_type=out_type, mesh=mesh,
                     scratch_types=scratch,
                     compiler_params=pltpu.CompilerParams(
                         use_tc_tiling_on_sc=False))


_DOT = functools.partial(
    lax.dot_general,
    dimension_numbers=(((1,), (1,)), ((), ())),
    preferred_element_type=jnp.float32,
)

_BR = 256


def _tc_rpath(h, wr, bl, d_in, d_out):
    def body(h_ref, wr_ref, bl_ref, xr_ref):
        xr_ref[...] = _DOT(h_ref[...], wr_ref[...]) + bl_ref[...]

    return pl.pallas_call(
        body,
        grid=(NPAD // _BR,),
        in_specs=[
            pl.BlockSpec((_BR, d_in), lambda i: (i, 0)),
            pl.BlockSpec((d_out, d_in), lambda i: (0, 0)),
            pl.BlockSpec((1, d_out), lambda i: (0, 0)),
        ],
        out_specs=pl.BlockSpec((_BR, d_out), lambda i: (i, 0)),
        out_shape=jax.ShapeDtypeStruct((NPAD, d_out), jnp.float32),
    )(h, wr, bl)


def _tc_rpath2(h1, wr, bl):
    def body(h1_ref, wr_ref, bl_ref, xr_ref):
        h1f = jnp.concatenate([h1_ref[q] for q in range(4)], axis=1)
        xr_ref[...] = _DOT(h1f, wr_ref[...]) + bl_ref[...]

    return pl.pallas_call(
        body,
        grid=(NPAD // _BR,),
        in_specs=[
            pl.BlockSpec((4, _BR, 64), lambda i: (0, i, 0)),
            pl.BlockSpec((512, 256), lambda i: (0, 0)),
            pl.BlockSpec((1, 512), lambda i: (0, 0)),
        ],
        out_specs=pl.BlockSpec((_BR, 512), lambda i: (i, 0)),
        out_shape=jax.ShapeDtypeStruct((NPAD, 512), jnp.float32),
    )(h1, wr, bl)


def _tc_layer1(acc, cnt, xr, wl):
    def body(acc_ref, cnt_ref, xr_ref, wl_ref, h1_ref):
        agg = jnp.concatenate([acc_ref[0], acc_ref[1]], axis=1)
        agg = agg / jnp.maximum(cnt_ref[:, 0:1], 1.0)
        h = _DOT(agg, wl_ref[...]) + xr_ref[...]
        h = jnp.maximum(h, 0.0)
        for q in range(4):
            h1_ref[q] = h[:, q * 64:(q + 1) * 64]

    return pl.pallas_call(
        body,
        grid=(NPAD // _BR,),
        in_specs=[
            pl.BlockSpec((2, _BR, 64), lambda i: (0, i, 0)),
            pl.BlockSpec((_BR, 8), lambda i: (i, 0)),
            pl.BlockSpec((_BR, 256), lambda i: (i, 0)),
            pl.BlockSpec((256, 128), lambda i: (0, 0)),
        ],
        out_specs=pl.BlockSpec((4, _BR, 64), lambda i: (0, i, 0)),
        out_shape=jax.ShapeDtypeStruct((4, NPAD, 64), jnp.float32),
    )(acc, cnt, xr, wl)


def _tc_layer2(acca, accb, cnt, xr, wl, w3l):
    def body(acca_ref, accb_ref, cnt_ref, xr_ref, wl_ref, w3_ref,
             h2_ref, y3_ref):
        agg = jnp.concatenate([acca_ref[0], acca_ref[1],
                               accb_ref[0], accb_ref[1]], axis=1)
        agg = agg / jnp.maximum(cnt_ref[:, 0:1], 1.0)
        h = _DOT(agg, wl_ref[...]) + xr_ref[...]
        h = jnp.maximum(h, 0.0)
        h2_ref[...] = h
        y3_ref[0] = _DOT(h, w3_ref[0])
        y3_ref[1] = _DOT(h, w3_ref[1])

    return pl.pallas_call(
        body,
        grid=(NPAD // _BR,),
        in_specs=[
            pl.BlockSpec((2, _BR, 64), lambda i: (0, i, 0)),
            pl.BlockSpec((2, _BR, 64), lambda i: (0, i, 0)),
            pl.BlockSpec((_BR, 8), lambda i: (i, 0)),
            pl.BlockSpec((_BR, 512), lambda i: (i, 0)),
            pl.BlockSpec((512, 256), lambda i: (0, 0)),
            pl.BlockSpec((2, 64, 512), lambda i: (0, 0, 0)),
        ],
        out_specs=[
            pl.BlockSpec((_BR, 512), lambda i: (i, 0)),
            pl.BlockSpec((2, _BR, 64), lambda i: (0, i, 0)),
        ],
        out_shape=[
            jax.ShapeDtypeStruct((NPAD, 512), jnp.float32),
            jax.ShapeDtypeStruct((2, NPAD, 64), jnp.float32),
        ],
    )(acca, accb, cnt, xr, wl, w3l)


def _tc_layer3(acc, cnt, xr):
    def body(acc_ref, cnt_ref, xr_ref, out_ref):
        agg = jnp.concatenate([acc_ref[0], acc_ref[1]], axis=1)
        agg = agg / jnp.maximum(cnt_ref[:, 0:1], 1.0)
        out_ref[...] = agg + xr_ref[...]

    return pl.pallas_call(
        body,
        grid=(NPAD // _BR,),
        in_specs=[
            pl.BlockSpec((2, _BR, 64), lambda i: (0, i, 0)),
            pl.BlockSpec((_BR, 8), lambda i: (i, 0)),
            pl.BlockSpec((_BR, 128), lambda i: (i, 0)),
        ],
        out_specs=pl.BlockSpec((_BR, 128), lambda i: (i, 0)),
        out_shape=jax.ShapeDtypeStruct((NPAD, 128), jnp.float32),
    )(acc, cnt, xr)


def kernel(x, edge_index, W1_l, b1_l, W1_r, W2_l, b2_l, W2_r, W3_l, b3_l, W3_r):
    x = x.astype(jnp.float32)
    src = edge_index[0].astype(jnp.int32)
    dst = edge_index[1].astype(jnp.int32)

    npad_e = EPAD - E
    pad_ar = jnp.arange(npad_e, dtype=jnp.int32)
    src_pad = jnp.concatenate([src, pad_ar % N])
    dst_pad = jnp.concatenate([dst, N + pad_ar % (NPAD - N)])
    src_t = src_pad.reshape(NTILE, NCHUNK, CHUNK)
    dst_t = dst_pad.reshape(NTILE, NCHUNK, CHUNK)

    zeros64 = jnp.zeros((NPAD, 64), jnp.float32)
    zeros128 = jnp.zeros((NPAD, 128), jnp.float32)
    zeros8 = jnp.zeros((NPAD, 8), jnp.float32)
    ones8 = jnp.ones((CHUNK, 8), jnp.float32)

    xpad = jnp.zeros((NPAD, D_IN), jnp.float32).at[:N].set(x)
    t1 = jnp.concatenate([xpad[:, :64], xpad[:, 64:]], axis=0)

    agg64c = _make_agg(True)
    agg64 = _make_agg(False)

    acc1, cnt = agg64c(t1, src_t, dst_t, zeros64, ones8, zeros8)
    xr1 = _tc_rpath(xpad, W1_r, b1_l.reshape(1, -1), 128, 256)
    h1 = _tc_layer1(acc1.reshape(2, NPAD, 64), cnt, xr1, W1_l)

    acc2a = agg64(h1[0:2].reshape(2 * NPAD, 64), src_t, dst_t, zeros64)
    acc2b = agg64(h1[2:4].reshape(2 * NPAD, 64), src_t, dst_t, zeros64)
    xr2 = _tc_rpath2(h1, W2_r, b2_l.reshape(1, -1))
    h2, y3 = _tc_layer2(acc2a.reshape(2, NPAD, 64), acc2b.reshape(2, NPAD, 64),
                        cnt, xr2, W2_l, W3_l.reshape(2, 64, 512))

    acc3 = agg64(y3.reshape(2 * NPAD, 64), src_t, dst_t, zeros64)
    xr3 = _tc_rpath(h2, W3_r, b3_l.reshape(1, -1), 512, 128)
    out = _tc_layer3(acc3.reshape(2, NPAD, 64), cnt, xr3)
    return out[:N]


# speedup vs baseline: 1.2684x; 1.2684x over previous
import functools

import jax
import jax.numpy as jnp
from jax import lax
from jax.experimental import pallas as pl
from jax.experimental.pallas import tpu as pltpu
from jax.experimental.pallas import tpu_sc as plsc

N = 10000
E = 320000
NPAD = 10240
D_IN = 128

NTILE = 16
NCORE = 2
CHUNK = 128
W = 128
GRP = 8
NGRP = 10
NCHUNK = GRP * NGRP
EPT = NCHUNK * CHUNK
EPAD = EPT * NTILE * NCORE
ROWS_PT = NPAD // NTILE


def _make_agg(with_counts):
    mesh = plsc.VectorSubcoreMesh(core_axis_name="c", subcore_axis_name="s")
    nbuf = 2
    npre = 1
    out_type = [jax.ShapeDtypeStruct((NCORE * NPAD, W), jnp.float32)]
    scratch = [
        pltpu.VMEM_SHARED((NPAD, W), jnp.float32),
        pltpu.VMEM((2, GRP, CHUNK), jnp.int32),
        pltpu.VMEM((2, GRP, CHUNK), jnp.int32),
        [pltpu.VMEM((CHUNK, W), jnp.float32) for _ in range(nbuf)],
        [pltpu.SemaphoreType.DMA for _ in range(nbuf)],
        [pltpu.SemaphoreType.DMA for _ in range(nbuf)],
        pltpu.SemaphoreType.DMA,
    ]
    if with_counts:
        out_type.append(jax.ShapeDtypeStruct((NCORE * NPAD, 8), jnp.float32))
        scratch += [
            pltpu.VMEM_SHARED((NPAD, 8), jnp.float32),
            pltpu.VMEM((CHUNK, 8), jnp.float32),
            pltpu.SemaphoreType.DMA,
        ]

    def body_common(table, srct, dstt, zeros_w, out, acc, sidx, didx,
                    bufs, gsems, ssems, isem, c, t, cnt_stuff):
        r0 = t * ROWS_PT
        pltpu.sync_copy(zeros_w.at[pl.ds(r0, ROWS_PT)],
                        acc.at[pl.ds(r0, ROWS_PT)])
        if cnt_stuff is not None:
            ones8, zeros8, cnt_out, cntacc, onesb, csem = cnt_stuff
            pltpu.sync_copy(zeros8.at[pl.ds(r0, ROWS_PT)],
                            cntacc.at[pl.ds(r0, ROWS_PT)])
            pltpu.sync_copy(ones8, onesb)

        def gather_start(slot, j, b):
            pltpu.async_copy(table.at[sidx.at[slot, j]], bufs[b], gsems[b])

        def gather_wait(b):
            pltpu.make_async_copy(table.at[sidx.at[0, 0]], bufs[b],
                                  gsems[b]).wait()

        def scatter_start(slot, j, b):
            pltpu.async_copy(bufs[b], acc.at[didx.at[slot, j]], ssems[b],
                             add=True)

        def scatter_wait(b):
            pltpu.make_async_copy(bufs[b], acc.at[didx.at[0, 0]],
                                  ssems[b]).wait()

        def count_start(slot, j):
            pltpu.async_copy(onesb, cntacc.at[didx.at[slot, j]], csem,
                             add=True)

        def count_wait():
            pltpu.make_async_copy(onesb, cntacc.at[didx.at[0, 0]],
                                  csem).wait()

        def idx_start(g):
            off = jnp.minimum(g, NGRP - 1) * GRP
            slot = lax.rem(g, 2)
            pltpu.async_copy(srct.at[c, t, pl.ds(off, GRP)],
                             sidx.at[slot], isem)
            pltpu.async_copy(dstt.at[c, t, pl.ds(off, GRP)],
                             didx.at[slot], isem)

        def idx_wait(g):
            slot = lax.rem(g, 2)
            pltpu.make_async_copy(srct.at[c, t, pl.ds(0, GRP)],
                                  sidx.at[slot], isem).wait()
            pltpu.make_async_copy(dstt.at[c, t, pl.ds(0, GRP)],
                                  didx.at[slot], isem).wait()

        def run_group(slot, nslot, first):
            for j in range(GRP):
                b = j % nbuf
                gather_wait(b)
                scatter_start(slot, j, b)
                if cnt_stuff is not None:
                    if not (first and j == 0):
                        count_wait()
                    count_start(slot, j)
                nj = j + npre
                if nj < GRP:
                    b2 = nj % nbuf
                    if not (first and nj < nbuf):
                        scatter_wait(b2)
                    gather_start(slot, nj, b2)
            for j in range(npre):
                b2 = j % nbuf
                scatter_wait(b2)
                gather_start(nslot, j, b2)

        idx_start(0)
        idx_wait(0)
        for j in range(npre):
            gather_start(0, j, j)
        plsc.subcore_barrier()

        idx_start(1)
        run_group(0, 1, True)
        idx_wait(1)

        def group(g, carry):
            idx_start(g + 1)
            run_group(lax.rem(g, 2), lax.rem(g + 1, 2), False)
            idx_wait(g + 1)
            return carry

        lax.fori_loop(1, NGRP, group, 0)
        for j in range(npre):
            gather_wait(j % nbuf)
        for k in range(nbuf - npre):
            scatter_wait((GRP - (nbuf - npre) + k) % nbuf)
        if cnt_stuff is not None:
            count_wait()
        plsc.subcore_barrier()
        pltpu.sync_copy(acc.at[pl.ds(r0, ROWS_PT)],
                        out.at[pl.ds(c * NPAD + r0, ROWS_PT)])
        if cnt_stuff is not None:
            pltpu.sync_copy(cntacc.at[pl.ds(r0, ROWS_PT)],
                            cnt_out.at[pl.ds(c * NPAD + r0, ROWS_PT)])

    if with_counts:
        def body(table, srct, dstt, zeros_w, ones8, zeros8, out, cnt_out,
                 acc, sidx, didx, bufs, gsems, ssems, isem,
                 cntacc, onesb, csem):
            c = lax.axis_index("c")
            t = lax.axis_index("s")
            body_common(table, srct, dstt, zeros_w, out, acc, sidx,
                        didx, bufs, gsems, ssems, isem, c, t,
                        (ones8, zeros8, cnt_out, cntacc, onesb, csem))
    else:
        def body(table, srct, dstt, zeros_w, out,
                 acc, sidx, didx, bufs, gsems, ssems, isem):
            c = lax.axis_index("c")
            t = lax.axis_index("s")
            body_common(table, srct, dstt, zeros_w, out, acc, sidx,
                        didx, bufs, gsems, ssems, isem, c, t, None)

    out_type = tuple(out_type) if with_counts else out_type[0]
    return pl.kernel(body, out_type=out_type, mesh=mesh,
                     scratch_types=scratch,
                     compiler_params=pltpu.CompilerParams(
                         use_tc_tiling_on_sc=False))


_DOT = functools.partial(
    lax.dot_general,
    dimension_numbers=(((1,), (1,)), ((), ())),
    preferred_element_type=jnp.float32,
)

_BR = 256


def _tc_rpath(h, wr, bl, d_in, d_out):
    def body(h_ref, wr_ref, bl_ref, xr_ref):
        xr_ref[...] = _DOT(h_ref[...], wr_ref[...]) + bl_ref[...]

    return pl.pallas_call(
        body,
        grid=(NPAD // _BR,),
        in_specs=[
            pl.BlockSpec((_BR, d_in), lambda i: (i, 0)),
            pl.BlockSpec((d_out, d_in), lambda i: (0, 0)),
            pl.BlockSpec((1, d_out), lambda i: (0, 0)),
        ],
        out_specs=pl.BlockSpec((_BR, d_out), lambda i: (i, 0)),
        out_shape=jax.ShapeDtypeStruct((NPAD, d_out), jnp.float32),
    )(h, wr, bl)


def _tc_rpath2(h1, wr, bl):
    def body(h1_ref, wr_ref, bl_ref, xr_ref):
        h1f = jnp.concatenate([h1_ref[0], h1_ref[1]], axis=1)
        xr_ref[...] = _DOT(h1f, wr_ref[...]) + bl_ref[...]

    return pl.pallas_call(
        body,
        grid=(NPAD // _BR,),
        in_specs=[
            pl.BlockSpec((2, _BR, 128), lambda i: (0, i, 0)),
            pl.BlockSpec((512, 256), lambda i: (0, 0)),
            pl.BlockSpec((1, 512), lambda i: (0, 0)),
        ],
        out_specs=pl.BlockSpec((_BR, 512), lambda i: (i, 0)),
        out_shape=jax.ShapeDtypeStruct((NPAD, 512), jnp.float32),
    )(h1, wr, bl)


def _tc_layer1(acc, cnt, xr, wl):
    def body(acc_ref, cnt_ref, xr_ref, wl_ref, h1_ref):
        agg = acc_ref[0] + acc_ref[1]
        deg = cnt_ref[0, :, 0:1] + cnt_ref[1, :, 0:1]
        agg = agg / jnp.maximum(deg, 1.0)
        h = _DOT(agg, wl_ref[...]) + xr_ref[...]
        h = jnp.maximum(h, 0.0)
        h1_ref[0] = h[:, :128]
        h1_ref[1] = h[:, 128:]

    return pl.pallas_call(
        body,
        grid=(NPAD // _BR,),
        in_specs=[
            pl.BlockSpec((2, _BR, 128), lambda i: (0, i, 0)),
            pl.BlockSpec((2, _BR, 8), lambda i: (0, i, 0)),
            pl.BlockSpec((_BR, 256), lambda i: (i, 0)),
            pl.BlockSpec((256, 128), lambda i: (0, 0)),
        ],
        out_specs=pl.BlockSpec((2, _BR, 128), lambda i: (0, i, 0)),
        out_shape=jax.ShapeDtypeStruct((2, NPAD, 128), jnp.float32),
    )(acc, cnt, xr, wl)


def _tc_layer2(acca, accb, cnt, xr, wl, w3l):
    def body(acca_ref, accb_ref, cnt_ref, xr_ref, wl_ref, w3_ref,
             h2_ref, y3_ref):
        agg = jnp.concatenate([acca_ref[0] + acca_ref[1],
                               accb_ref[0] + accb_ref[1]], axis=1)
        deg = cnt_ref[0, :, 0:1] + cnt_ref[1, :, 0:1]
        agg = agg / jnp.maximum(deg, 1.0)
        h = _DOT(agg, wl_ref[...]) + xr_ref[...]
        h = jnp.maximum(h, 0.0)
        h2_ref[...] = h
        y3_ref[...] = _DOT(h, w3_ref[...])

    return pl.pallas_call(
        body,
        grid=(NPAD // _BR,),
        in_specs=[
            pl.BlockSpec((2, _BR, 128), lambda i: (0, i, 0)),
            pl.BlockSpec((2, _BR, 128), lambda i: (0, i, 0)),
            pl.BlockSpec((2, _BR, 8), lambda i: (0, i, 0)),
            pl.BlockSpec((_BR, 512), lambda i: (i, 0)),
            pl.BlockSpec((512, 256), lambda i: (0, 0)),
            pl.BlockSpec((128, 512), lambda i: (0, 0)),
        ],
        out_specs=[
            pl.BlockSpec((_BR, 512), lambda i: (i, 0)),
            pl.BlockSpec((_BR, 128), lambda i: (i, 0)),
        ],
        out_shape=[
            jax.ShapeDtypeStruct((NPAD, 512), jnp.float32),
            jax.ShapeDtypeStruct((NPAD, 128), jnp.float32),
        ],
    )(acca, accb, cnt, xr, wl, w3l)


def _tc_layer3(acc, cnt, xr):
    def body(acc_ref, cnt_ref, xr_ref, out_ref):
        agg = acc_ref[0] + acc_ref[1]
        deg = cnt_ref[0, :, 0:1] + cnt_ref[1, :, 0:1]
        agg = agg / jnp.maximum(deg, 1.0)
        out_ref[...] = agg + xr_ref[...]

    return pl.pallas_call(
        body,
        grid=(NPAD // _BR,),
        in_specs=[
            pl.BlockSpec((2, _BR, 128), lambda i: (0, i, 0)),
            pl.BlockSpec((2, _BR, 8), lambda i: (0, i, 0)),
            pl.BlockSpec((_BR, 128), lambda i: (i, 0)),
        ],
        out_specs=pl.BlockSpec((_BR, 128), lambda i: (i, 0)),
        out_shape=jax.ShapeDtypeStruct((NPAD, 128), jnp.float32),
    )(acc, cnt, xr)


def kernel(x, edge_index, W1_l, b1_l, W1_r, W2_l, b2_l, W2_r, W3_l, b3_l, W3_r):
    x = x.astype(jnp.float32)
    src = edge_index[0].astype(jnp.int32)
    dst = edge_index[1].astype(jnp.int32)

    npad_e = EPAD - E
    pad_ar = jnp.arange(npad_e, dtype=jnp.int32)
    src_pad = jnp.concatenate([src, pad_ar % N])
    dst_pad = jnp.concatenate([dst, N + pad_ar % (NPAD - N)])
    src_t = src_pad.reshape(NCORE, NTILE, NCHUNK, CHUNK)
    dst_t = dst_pad.reshape(NCORE, NTILE, NCHUNK, CHUNK)

    zeros128 = jnp.zeros((NPAD, 128), jnp.float32)
    zeros8 = jnp.zeros((NPAD, 8), jnp.float32)
    ones8 = jnp.ones((CHUNK, 8), jnp.float32)

    xpad = jnp.zeros((NPAD, D_IN), jnp.float32).at[:N].set(x)

    agg128c = _make_agg(True)
    agg128 = _make_agg(False)

    acc1, cnt = agg128c(xpad, src_t, dst_t, zeros128, ones8, zeros8)
    cnt = cnt.reshape(2, NPAD, 8)
    xr1 = _tc_rpath(xpad, W1_r, b1_l.reshape(1, -1), 128, 256)
    h1 = _tc_layer1(acc1.reshape(2, NPAD, 128), cnt, xr1, W1_l)

    acc2a = agg128(h1[0], src_t, dst_t, zeros128)
    acc2b = agg128(h1[1], src_t, dst_t, zeros128)
    xr2 = _tc_rpath2(h1, W2_r, b2_l.reshape(1, -1))
    h2, y3 = _tc_layer2(acc2a.reshape(2, NPAD, 128), acc2b.reshape(2, NPAD, 128),
                        cnt, xr2, W2_l, W3_l)

    acc3 = agg128(y3, src_t, dst_t, zeros128)
    xr3 = _tc_rpath(h2, W3_r, b3_l.reshape(1, -1), 512, 128)
    out = _tc_layer3(acc3.reshape(2, NPAD, 128), cnt, xr3)
    return out[:N]


# speedup vs baseline: 1.2722x; 1.0030x over previous
import functools

import jax
import jax.numpy as jnp
from jax import lax
from jax.experimental import pallas as pl
from jax.experimental.pallas import tpu as pltpu
from jax.experimental.pallas import tpu_sc as plsc

N = 10000
E = 320000
NPAD = 10240
D_IN = 128

NTILE = 16
NCORE = 2
CHUNK = 128
W = 128
GRP = 8
NGRP = 10
NCHUNK = GRP * NGRP
EPT = NCHUNK * CHUNK
EPAD = EPT * NTILE * NCORE
ROWS_PT = NPAD // NTILE


def _make_agg(with_counts):
    mesh = plsc.VectorSubcoreMesh(core_axis_name="c", subcore_axis_name="s")
    nbuf = 2
    npre = 1
    out_type = [jax.ShapeDtypeStruct((NCORE * NPAD, W), jnp.float32)]
    scratch = [
        pltpu.VMEM_SHARED((NPAD, W), jnp.float32),
        pltpu.VMEM((2, GRP, CHUNK), jnp.int32),
        pltpu.VMEM((2, GRP, CHUNK), jnp.int32),
        [pltpu.VMEM((CHUNK, W), jnp.float32) for _ in range(nbuf)],
        [pltpu.SemaphoreType.DMA for _ in range(nbuf)],
        [pltpu.SemaphoreType.DMA for _ in range(nbuf)],
        pltpu.SemaphoreType.DMA,
    ]
    if with_counts:
        out_type.append(jax.ShapeDtypeStruct((NCORE * NPAD, 8), jnp.float32))
        scratch += [
            pltpu.VMEM_SHARED((NPAD, 8), jnp.float32),
            pltpu.VMEM((CHUNK, 8), jnp.float32),
            pltpu.SemaphoreType.DMA,
        ]

    def body_common(table, srct, dstt, zeros_w, out, acc, sidx, didx,
                    bufs, gsems, ssems, isem, c, t, cnt_stuff):
        r0 = t * ROWS_PT
        pltpu.sync_copy(zeros_w.at[pl.ds(r0, ROWS_PT)],
                        acc.at[pl.ds(r0, ROWS_PT)])
        if cnt_stuff is not None:
            ones8, zeros8, cnt_out, cntacc, onesb, csem = cnt_stuff
            pltpu.sync_copy(zeros8.at[pl.ds(r0, ROWS_PT)],
                            cntacc.at[pl.ds(r0, ROWS_PT)])
            pltpu.sync_copy(ones8, onesb)

        def gather_start(slot, j, b):
            pltpu.async_copy(table.at[sidx.at[slot, j]], bufs[b], gsems[b])

        def gather_wait(b):
            pltpu.make_async_copy(table.at[sidx.at[0, 0]], bufs[b],
                                  gsems[b]).wait()

        def scatter_start(slot, j, b):
            pltpu.async_copy(bufs[b], acc.at[didx.at[slot, j]], ssems[b],
                             add=True)

        def scatter_wait(b):
            pltpu.make_async_copy(bufs[b], acc.at[didx.at[0, 0]],
                                  ssems[b]).wait()

        def count_start(slot, j):
            pltpu.async_copy(onesb, cntacc.at[didx.at[slot, j]], csem,
                             add=True)

        def count_wait():
            pltpu.make_async_copy(onesb, cntacc.at[didx.at[0, 0]],
                                  csem).wait()

        def idx_start(g):
            off = jnp.minimum(g, NGRP - 1) * GRP
            slot = lax.rem(g, 2)
            pltpu.async_copy(srct.at[c, t, pl.ds(off, GRP)],
                             sidx.at[slot], isem)
            pltpu.async_copy(dstt.at[c, t, pl.ds(off, GRP)],
                             didx.at[slot], isem)

        def idx_wait(g):
            slot = lax.rem(g, 2)
            pltpu.make_async_copy(srct.at[c, t, pl.ds(0, GRP)],
                                  sidx.at[slot], isem).wait()
            pltpu.make_async_copy(dstt.at[c, t, pl.ds(0, GRP)],
                                  didx.at[slot], isem).wait()

        def run_group(slot, nslot, first):
            for j in range(GRP):
                b = j % nbuf
                gather_wait(b)
                scatter_start(slot, j, b)
                if cnt_stuff is not None:
                    if not (first and j == 0):
                        count_wait()
                    count_start(slot, j)
                nj = j + npre
                if nj < GRP:
                    b2 = nj % nbuf
                    if not (first and nj < nbuf):
                        scatter_wait(b2)
                    gather_start(slot, nj, b2)
            for j in range(npre):
                b2 = j % nbuf
                scatter_wait(b2)
                gather_start(nslot, j, b2)

        idx_start(0)
        idx_wait(0)
        for j in range(npre):
            gather_start(0, j, j)
        plsc.subcore_barrier()

        idx_start(1)
        run_group(0, 1, True)
        idx_wait(1)

        def group(g, carry):
            idx_start(g + 1)
            run_group(lax.rem(g, 2), lax.rem(g + 1, 2), False)
            idx_wait(g + 1)
            return carry

        lax.fori_loop(1, NGRP, group, 0)
        for j in range(npre):
            gather_wait(j % nbuf)
        for k in range(nbuf - npre):
            scatter_wait((GRP - (nbuf - npre) + k) % nbuf)
        if cnt_stuff is not None:
            count_wait()
        plsc.subcore_barrier()
        pltpu.sync_copy(acc.at[pl.ds(r0, ROWS_PT)],
                        out.at[pl.ds(c * NPAD + r0, ROWS_PT)])
        if cnt_stuff is not None:
            pltpu.sync_copy(cntacc.at[pl.ds(r0, ROWS_PT)],
                            cnt_out.at[pl.ds(c * NPAD + r0, ROWS_PT)])

    if with_counts:
        def body(table, srct, dstt, zeros_w, ones8, zeros8, out, cnt_out,
                 acc, sidx, didx, bufs, gsems, ssems, isem,
                 cntacc, onesb, csem):
            c = lax.axis_index("c")
            t = lax.axis_index("s")
            body_common(table, srct, dstt, zeros_w, out, acc, sidx,
                        didx, bufs, gsems, ssems, isem, c, t,
                        (ones8, zeros8, cnt_out, cntacc, onesb, csem))
    else:
        def body(table, srct, dstt, zeros_w, out,
                 acc, sidx, didx, bufs, gsems, ssems, isem):
            c = lax.axis_index("c")
            t = lax.axis_index("s")
            body_common(table, srct, dstt, zeros_w, out, acc, sidx,
                        didx, bufs, gsems, ssems, isem, c, t, None)

    out_type = tuple(out_type) if with_counts else out_type[0]
    return pl.kernel(body, out_type=out_type, mesh=mesh,
                     scratch_types=scratch,
                     compiler_params=pltpu.CompilerParams(
                         use_tc_tiling_on_sc=False))


_DOT = functools.partial(
    lax.dot_general,
    dimension_numbers=(((1,), (1,)), ((), ())),
    preferred_element_type=jnp.float32,
)

_BR = 256


def _tc_layer1(acc, cnt, x, wl, wr, bl):
    def body(acc_ref, cnt_ref, x_ref, wl_ref, wr_ref, bl_ref, h1_ref):
        agg = acc_ref[0] + acc_ref[1]
        deg = cnt_ref[0, :, 0:1] + cnt_ref[1, :, 0:1]
        agg = agg / jnp.maximum(deg, 1.0)
        h = (_DOT(agg, wl_ref[...]) + _DOT(x_ref[...], wr_ref[...])
             + bl_ref[...])
        h = jnp.maximum(h, 0.0)
        h1_ref[0] = h[:, :128]
        h1_ref[1] = h[:, 128:]

    return pl.pallas_call(
        body,
        grid=(NPAD // _BR,),
        in_specs=[
            pl.BlockSpec((2, _BR, 128), lambda i: (0, i, 0)),
            pl.BlockSpec((2, _BR, 8), lambda i: (0, i, 0)),
            pl.BlockSpec((_BR, 128), lambda i: (i, 0)),
            pl.BlockSpec((256, 128), lambda i: (0, 0)),
            pl.BlockSpec((256, 128), lambda i: (0, 0)),
            pl.BlockSpec((1, 256), lambda i: (0, 0)),
        ],
        out_specs=pl.BlockSpec((2, _BR, 128), lambda i: (0, i, 0)),
        out_shape=jax.ShapeDtypeStruct((2, NPAD, 128), jnp.float32),
    )(acc, cnt, x, wl, wr, bl)


def _tc_layer2(acca, accb, cnt, h1, wl, wr, bl, w3l):
    def body(acca_ref, accb_ref, cnt_ref, h1_ref, wl_ref, wr_ref, bl_ref,
             w3_ref, h2_ref, y3_ref):
        agg = jnp.concatenate([acca_ref[0] + acca_ref[1],
                               accb_ref[0] + accb_ref[1]], axis=1)
        deg = cnt_ref[0, :, 0:1] + cnt_ref[1, :, 0:1]
        agg = agg / jnp.maximum(deg, 1.0)
        h1f = jnp.concatenate([h1_ref[0], h1_ref[1]], axis=1)
        h = (_DOT(agg, wl_ref[...]) + _DOT(h1f, wr_ref[...])
             + bl_ref[...])
        h = jnp.maximum(h, 0.0)
        h2_ref[...] = h
        y3_ref[...] = _DOT(h, w3_ref[...])

    return pl.pallas_call(
        body,
        grid=(NPAD // _BR,),
        in_specs=[
            pl.BlockSpec((2, _BR, 128), lambda i: (0, i, 0)),
            pl.BlockSpec((2, _BR, 128), lambda i: (0, i, 0)),
            pl.BlockSpec((2, _BR, 8), lambda i: (0, i, 0)),
            pl.BlockSpec((2, _BR, 128), lambda i: (0, i, 0)),
            pl.BlockSpec((512, 256), lambda i: (0, 0)),
            pl.BlockSpec((512, 256), lambda i: (0, 0)),
            pl.BlockSpec((1, 512), lambda i: (0, 0)),
            pl.BlockSpec((128, 512), lambda i: (0, 0)),
        ],
        out_specs=[
            pl.BlockSpec((_BR, 512), lambda i: (i, 0)),
            pl.BlockSpec((_BR, 128), lambda i: (i, 0)),
        ],
        out_shape=[
            jax.ShapeDtypeStruct((NPAD, 512), jnp.float32),
            jax.ShapeDtypeStruct((NPAD, 128), jnp.float32),
        ],
    )(acca, accb, cnt, h1, wl, wr, bl, w3l)


def _tc_layer3(acc, cnt, h2, wr, bl):
    def body(acc_ref, cnt_ref, h2_ref, wr_ref, bl_ref, out_ref):
        agg = acc_ref[0] + acc_ref[1]
        deg = cnt_ref[0, :, 0:1] + cnt_ref[1, :, 0:1]
        agg = agg / jnp.maximum(deg, 1.0)
        out_ref[...] = (agg + _DOT(h2_ref[...], wr_ref[...])
                        + bl_ref[...])

    return pl.pallas_call(
        body,
        grid=(NPAD // _BR,),
        in_specs=[
            pl.BlockSpec((2, _BR, 128), lambda i: (0, i, 0)),
            pl.BlockSpec((2, _BR, 8), lambda i: (0, i, 0)),
            pl.BlockSpec((_BR, 512), lambda i: (i, 0)),
            pl.BlockSpec((128, 512), lambda i: (0, 0)),
            pl.BlockSpec((1, 128), lambda i: (0, 0)),
        ],
        out_specs=pl.BlockSpec((_BR, 128), lambda i: (i, 0)),
        out_shape=jax.ShapeDtypeStruct((NPAD, 128), jnp.float32),
    )(acc, cnt, h2, wr, bl)


def kernel(x, edge_index, W1_l, b1_l, W1_r, W2_l, b2_l, W2_r, W3_l, b3_l, W3_r):
    x = x.astype(jnp.float32)
    src = edge_index[0].astype(jnp.int32)
    dst = edge_index[1].astype(jnp.int32)

    npad_e = EPAD - E
    pad_ar = jnp.arange(npad_e, dtype=jnp.int32)
    src_pad = jnp.concatenate([src, pad_ar % N])
    dst_pad = jnp.concatenate([dst, N + pad_ar % (NPAD - N)])
    src_t = src_pad.reshape(NCORE, NTILE, NCHUNK, CHUNK)
    dst_t = dst_pad.reshape(NCORE, NTILE, NCHUNK, CHUNK)

    zeros128 = jnp.zeros((NPAD, 128), jnp.float32)
    zeros8 = jnp.zeros((NPAD, 8), jnp.float32)
    ones8 = jnp.ones((CHUNK, 8), jnp.float32)

    xpad = jnp.zeros((NPAD, D_IN), jnp.float32).at[:N].set(x)

    agg128c = _make_agg(True)
    agg128 = _make_agg(False)

    acc1, cnt = agg128c(xpad, src_t, dst_t, zeros128, ones8, zeros8)
    cnt = cnt.reshape(2, NPAD, 8)
    h1 = _tc_layer1(acc1.reshape(2, NPAD, 128), cnt, xpad, W1_l, W1_r,
                    b1_l.reshape(1, -1))

    acc2a = agg128(h1[0], src_t, dst_t, zeros128)
    acc2b = agg128(h1[1], src_t, dst_t, zeros128)
    h2, y3 = _tc_layer2(acc2a.reshape(2, NPAD, 128), acc2b.reshape(2, NPAD, 128),
                        cnt, h1, W2_l, W2_r, b2_l.reshape(1, -1), W3_l)

    acc3 = agg128(y3, src_t, dst_t, zeros128)
    out = _tc_layer3(acc3.reshape(2, NPAD, 128), cnt, h2, W3_r,
                     b3_l.reshape(1, -1))
    return out[:N]
